# Initial kernel scaffold; baseline (speedup 1.0000x reference)
import jax
import jax.numpy as jnp
from jax.experimental import pallas as pl


def kernel(fused_img, source_img1, source_img2, idx1, idx2):
    raise NotImplementedError("write your pallas kernel here")


# speedup vs baseline: 1.6707x; 1.6707x over previous
import functools

import jax
import jax.numpy as jnp
from jax.experimental import pallas as pl
from jax.experimental.pallas import tpu as pltpu

_SIGMA = 0.4
_SAMPLE = 10000
_EPS = 1e-10

_NP = 10240
_BR = 256
_BC = 128
_M = _NP // _BR
_PAD = 1e4
_C = -0.5 * 1.4426950408889634


def _rowsum_kernel(a_row_ref, b_row_ref, a_col_ref, b_col_ref,
                   oa_ref, ob_ref, oab_ref):
    ars = a_row_ref[0]
    brs = b_row_ref[0]
    acc_a = jnp.zeros((_BR, _BC), jnp.float32)
    acc_b = jnp.zeros((_BR, _BC), jnp.float32)
    acc_ab = jnp.zeros((_BR, _BC), jnp.float32)
    for c in range(_NP // _BC):
        ac = a_col_ref[0, 0:1, c * _BC:(c + 1) * _BC]
        bc = b_col_ref[0, 0:1, c * _BC:(c + 1) * _BC]
        da = ac - ars
        db = bc - brs
        ka = jnp.exp2(da * da * _C)
        kb = jnp.exp2(db * db * _C)
        acc_a = acc_a + ka
        acc_b = acc_b + kb
        acc_ab = acc_ab + ka * kb
    oa_ref[...] = jnp.sum(acc_a, axis=1, keepdims=True).reshape(1, _BR, 1)
    ob_ref[...] = jnp.sum(acc_b, axis=1, keepdims=True).reshape(1, _BR, 1)
    oab_ref[...] = jnp.sum(acc_ab, axis=1, keepdims=True).reshape(1, _BR, 1)


def _mi_kernel(sa_ref, sb_ref, sab_ref, o_ref):
    rows = jax.lax.broadcasted_iota(jnp.int32, (_NP // 128, 128), 0)
    lanes = jax.lax.broadcasted_iota(jnp.int32, (_NP // 128, 128), 1)
    mask = (rows * 128 + lanes) < _SAMPLE
    inv_n = jnp.float32(1.0 / _SAMPLE)

    def one_pair(p):
        rm_a = jnp.where(mask, sa_ref[p] * inv_n, 0.0)
        rm_b = jnp.where(mask, sb_ref[p] * inv_n, 0.0)
        rm_ab = jnp.where(mask, sab_ref[p] * inv_n, 0.0)
        p_a = rm_a / (jnp.sum(rm_a) + _EPS)
        p_b = rm_b / (jnp.sum(rm_b) + _EPS)
        p_ab = rm_ab / (jnp.sum(rm_ab) + _EPS)
        term = p_ab * jnp.log((p_ab + _EPS) / (p_a * p_b + _EPS))
        return jnp.sum(term)

    o_ref[...] = jnp.full((8, 128), -(one_pair(0) + one_pair(1)), jnp.float32)


@functools.partial(jax.jit, static_argnames=("interpret",))
def _mi_loss(a, b, interpret=False):
    a3 = a.reshape(2, _NP, 1)
    b3 = b.reshape(2, _NP, 1)
    ac3 = a.reshape(2, 1, _NP)
    bc3 = b.reshape(2, 1, _NP)
    row_spec = pl.BlockSpec((1, _BR, 1), lambda p, r: (p, r, 0))
    col_spec = pl.BlockSpec((1, 1, _NP), lambda p, r: (p, 0, 0))
    out_spec = pl.BlockSpec((1, _BR, 1), lambda p, r: (p, r, 0))
    sums = pl.pallas_call(
        _rowsum_kernel,
        grid=(2, _M),
        in_specs=[row_spec, row_spec, col_spec, col_spec],
        out_specs=[out_spec, out_spec, out_spec],
        out_shape=[jax.ShapeDtypeStruct((2, _NP, 1), jnp.float32)] * 3,
        compiler_params=pltpu.CompilerParams(
            dimension_semantics=("parallel", "arbitrary"),
        ),
        name="kde_rowsums",
        interpret=interpret,
    )(a3, b3, ac3, bc3)
    sa, sb, sab = (s.reshape(2, _NP // 128, 128) for s in sums)
    out = pl.pallas_call(
        _mi_kernel,
        out_shape=jax.ShapeDtypeStruct((8, 128), jnp.float32),
        name="kde_mi_reduce",
        interpret=interpret,
    )(sa, sb, sab)
    return out[0, 0]


def _prep(fused_img, source_img1, source_img2, idx1, idx2):
    f = fused_img.reshape(-1)
    s1 = source_img1.reshape(-1)
    s2 = source_img2.reshape(-1)
    scale = jnp.float32(1.0 / _SIGMA)
    a = jnp.stack([f[idx1], f[idx2]]) * scale
    b = jnp.stack([s1[idx1], s2[idx2]]) * scale
    pad = jnp.full((2, _NP - _SAMPLE), _PAD, jnp.float32)
    a = jnp.concatenate([a, pad], axis=1)
    b = jnp.concatenate([b, pad], axis=1)
    return a, b


def kernel(fused_img, source_img1, source_img2, idx1, idx2):
    a, b = _prep(fused_img, source_img1, source_img2, idx1, idx2)
    return _mi_loss(a, b)
